# baseline (device time: 225260 ns/iter reference)
import jax
import jax.numpy as jnp
from jax import lax
from jax.experimental import pallas as pl
from jax.experimental.pallas import tpu as pltpu

N_DEV = 16
M = 4096
N = 2048
CHUNK = M // N_DEV
HALF = N // 2
NSLOT = 4
SEG = 4
SROWS = CHUNK // SEG

RING = [0, 1, 5, 4, 8, 9, 13, 12, 15, 14, 10, 11, 7, 6, 2, 3]
POS = [0] * N_DEV
for _p, _idx in enumerate(RING):
    POS[_idx] = _p


def kernel(x, w_mat, scale_x, scale_w):
    my_idx = lax.axis_index("i")
    pos = jnp.asarray(POS, jnp.int32)[my_idx]
    ring = jnp.asarray(RING, jnp.int32)
    right = ring[(pos + 1) % N_DEV]
    left = ring[(pos - 1) % N_DEV]

    pos_s = jnp.reshape(pos, (1,)).astype(jnp.int32)
    right_s = jnp.reshape(right, (1,)).astype(jnp.int32)
    left_s = jnp.reshape(left, (1,)).astype(jnp.int32)
    scale = (scale_x * scale_w).astype(jnp.float32)

    def body(x_ref, w_ref, scale_ref, pos_ref, right_ref, left_ref,
             out_ref, rx_cw, rx_ccw, tx_cw, tx_ccw, ag_cw, ag_ccw,
             rs_send_cw, rs_recv_cw, rs_send_ccw, rs_recv_ccw,
             ag_send_cw, ag_recv_cw, ag_send_ccw, ag_recv_ccw,
             credit_cw, credit_ccw):
        my_pos = pos_ref[0]
        rgt = right_ref[0]
        lft = left_ref[0]

        D = [
            dict(tx=tx_cw, rx=rx_cw, ag=ag_cw, rss=rs_send_cw,
                 rsr=rs_recv_cw, ags=ag_send_cw, agr=ag_recv_cw,
                 credit=credit_cw, peer=rgt, up=lft,
                 cols=pl.ds(0, HALF), sgn=-1),
            dict(tx=tx_ccw, rx=rx_ccw, ag=ag_ccw, rss=rs_send_ccw,
                 rsr=rs_recv_ccw, ags=ag_send_ccw, agr=ag_recv_ccw,
                 credit=credit_ccw, peer=lft, up=rgt,
                 cols=pl.ds(HALF, HALF), sgn=+1),
        ]

        def rs_c(d, s):
            return (my_pos + d["sgn"] * s + 2 * N_DEV) % N_DEV

        def ag_c(d, s):
            return (my_pos - d["sgn"] + d["sgn"] * s + 2 * N_DEV) % N_DEV

        def seg(g):
            return pl.ds(g * SROWS, SROWS)

        def chunk_seg(c, g):
            return pl.ds(c * CHUNK + g * SROWS, SROWS)

        barrier_sem = pltpu.get_barrier_semaphore()
        pl.semaphore_signal(barrier_sem, inc=1, device_id=(lft,),
                            device_id_type=pl.DeviceIdType.MESH)
        pl.semaphore_signal(barrier_sem, inc=1, device_id=(rgt,),
                            device_id_type=pl.DeviceIdType.MESH)
        pl.semaphore_wait(barrier_sem, 2)

        for cb in range(N_DEV):
            acc = lax.dot_general(
                x_ref[pl.ds(cb * CHUNK, CHUNK), :], w_ref[:, :],
                dimension_numbers=(((1,), (0,)), ((), ())),
                preferred_element_type=jnp.int32,
            )
            out_ref[pl.ds(cb * CHUNK, CHUNK), :] = acc.astype(jnp.float32)

        rows0 = pl.ds(my_pos * CHUNK, CHUNK)
        tx_cw[0] = out_ref[rows0, pl.ds(0, HALF)].astype(jnp.bfloat16)
        tx_ccw[0] = out_ref[rows0, pl.ds(HALF, HALF)].astype(jnp.bfloat16)

        def rs_start(d, s, g):
            rd = pltpu.make_async_remote_copy(
                src_ref=d["tx"].at[s % 2, seg(g), :],
                dst_ref=d["rx"].at[s % NSLOT, seg(g), :],
                send_sem=d["rss"].at[s, g],
                recv_sem=d["rsr"].at[s, g],
                device_id=(d["peer"],),
                device_id_type=pl.DeviceIdType.MESH,
            )
            rd.start()
            return rd

        rs_desc = [[[None] * SEG for _ in range(N_DEV - 1)] for _ in D]
        for di, d in enumerate(D):
            for g in range(SEG):
                rs_desc[di][0][g] = rs_start(d, 0, g)

        for s in range(N_DEV - 1):
            slot = s % NSLOT
            for g in range(SEG):
                for di, d in enumerate(D):
                    rs_desc[di][s][g].wait_recv()
                    c_in = rs_c(d, s + 1)
                    rx_f32 = d["rx"][slot, seg(g), :].astype(jnp.float32)
                    if s < N_DEV - 2:
                        if g == 0 and s + 1 >= NSLOT:
                            pl.semaphore_wait(d["credit"], 1)
                        if s >= 1:
                            rs_desc[di][s - 1][g].wait_send()
                        new = out_ref[chunk_seg(c_in, g), d["cols"]] + rx_f32
                        d["tx"][(s + 1) % 2, seg(g), :] = (
                            new.astype(jnp.bfloat16))
                        rs_desc[di][s + 1][g] = rs_start(d, s + 1, g)
                    else:
                        fin = jnp.maximum(
                            (out_ref[chunk_seg(c_in, g), d["cols"]]
                             + rx_f32) * scale_ref[0], 0.0)
                        out_ref[chunk_seg(c_in, g), d["cols"]] = fin
                        d["ag"][c_in, seg(g), :] = fin.astype(jnp.bfloat16)
            if s <= N_DEV - 2 - NSLOT:
                for d in D:
                    pl.semaphore_signal(d["credit"], inc=1,
                                        device_id=(d["up"],),
                                        device_id_type=pl.DeviceIdType.MESH)
        for di, d in enumerate(D):
            for g in range(SEG):
                rs_desc[di][N_DEV - 3][g].wait_send()
                rs_desc[di][N_DEV - 2][g].wait_send()

        def ag_start(d, s, g):
            ref = d["ag"].at[ag_c(d, s), seg(g), :]
            rd = pltpu.make_async_remote_copy(
                src_ref=ref,
                dst_ref=ref,
                send_sem=d["ags"].at[s, g],
                recv_sem=d["agr"].at[s, g],
                device_id=(d["peer"],),
                device_id_type=pl.DeviceIdType.MESH,
            )
            rd.start()
            return rd

        ag_desc = [[[None] * SEG for _ in range(N_DEV - 1)] for _ in D]
        for di, d in enumerate(D):
            for g in range(SEG):
                ag_desc[di][0][g] = ag_start(d, 0, g)

        for s in range(N_DEV - 1):
            for g in range(SEG):
                for di, d in enumerate(D):
                    ag_desc[di][s][g].wait_recv()
                    c_in = ag_c(d, s + 1)
                    if s < N_DEV - 2:
                        ag_desc[di][s + 1][g] = ag_start(d, s + 1, g)
                    out_ref[chunk_seg(c_in, g), d["cols"]] = (
                        d["ag"][c_in, seg(g), :].astype(jnp.float32))
        for di, d in enumerate(D):
            for ss in range(N_DEV - 1):
                for g in range(SEG):
                    ag_desc[di][ss][g].wait_send()

    return pl.pallas_call(
        body,
        out_shape=jax.ShapeDtypeStruct((M, N), jnp.float32),
        in_specs=[
            pl.BlockSpec(memory_space=pltpu.VMEM),
            pl.BlockSpec(memory_space=pltpu.VMEM),
            pl.BlockSpec(memory_space=pltpu.SMEM),
            pl.BlockSpec(memory_space=pltpu.SMEM),
            pl.BlockSpec(memory_space=pltpu.SMEM),
            pl.BlockSpec(memory_space=pltpu.SMEM),
        ],
        out_specs=pl.BlockSpec(memory_space=pltpu.VMEM),
        scratch_shapes=[
            pltpu.VMEM((NSLOT, CHUNK, HALF), jnp.bfloat16),
            pltpu.VMEM((NSLOT, CHUNK, HALF), jnp.bfloat16),
            pltpu.VMEM((2, CHUNK, HALF), jnp.bfloat16),
            pltpu.VMEM((2, CHUNK, HALF), jnp.bfloat16),
            pltpu.VMEM((N_DEV, CHUNK, HALF), jnp.bfloat16),
            pltpu.VMEM((N_DEV, CHUNK, HALF), jnp.bfloat16),
            pltpu.SemaphoreType.DMA((N_DEV - 1, SEG)),
            pltpu.SemaphoreType.DMA((N_DEV - 1, SEG)),
            pltpu.SemaphoreType.DMA((N_DEV - 1, SEG)),
            pltpu.SemaphoreType.DMA((N_DEV - 1, SEG)),
            pltpu.SemaphoreType.DMA((N_DEV - 1, SEG)),
            pltpu.SemaphoreType.DMA((N_DEV - 1, SEG)),
            pltpu.SemaphoreType.DMA((N_DEV - 1, SEG)),
            pltpu.SemaphoreType.DMA((N_DEV - 1, SEG)),
            pltpu.SemaphoreType.REGULAR,
            pltpu.SemaphoreType.REGULAR,
        ],
        compiler_params=pltpu.CompilerParams(
            collective_id=0, vmem_limit_bytes=100 * 1024 * 1024
        ),
    )(x, w_mat, scale, pos_s, right_s, left_s)


# device time: 219207 ns/iter; 1.0276x vs baseline; 1.0276x over previous
import jax
import jax.numpy as jnp
from jax import lax
from jax.experimental import pallas as pl
from jax.experimental.pallas import tpu as pltpu

N_DEV = 16
M = 4096
N = 2048
CHUNK = M // N_DEV
HALF = N // 2
NSLOT = 4
SEG = 2
SROWS = CHUNK // SEG

RING = [0, 1, 5, 4, 8, 9, 13, 12, 15, 14, 10, 11, 7, 6, 2, 3]
POS = [0] * N_DEV
for _p, _idx in enumerate(RING):
    POS[_idx] = _p


def kernel(x, w_mat, scale_x, scale_w):
    my_idx = lax.axis_index("i")
    pos = jnp.asarray(POS, jnp.int32)[my_idx]
    ring = jnp.asarray(RING, jnp.int32)
    right = ring[(pos + 1) % N_DEV]
    left = ring[(pos - 1) % N_DEV]

    pos_s = jnp.reshape(pos, (1,)).astype(jnp.int32)
    right_s = jnp.reshape(right, (1,)).astype(jnp.int32)
    left_s = jnp.reshape(left, (1,)).astype(jnp.int32)
    scale = (scale_x * scale_w).astype(jnp.float32)

    def body(x_ref, w_ref, scale_ref, pos_ref, right_ref, left_ref,
             out_ref, rx_cw, rx_ccw, tx_cw, tx_ccw, ag_cw, ag_ccw,
             rs_send_cw, rs_recv_cw, rs_send_ccw, rs_recv_ccw,
             ag_send_cw, ag_recv_cw, ag_send_ccw, ag_recv_ccw,
             credit_cw, credit_ccw):
        my_pos = pos_ref[0]
        rgt = right_ref[0]
        lft = left_ref[0]

        D = [
            dict(tx=tx_cw, rx=rx_cw, ag=ag_cw, rss=rs_send_cw,
                 rsr=rs_recv_cw, ags=ag_send_cw, agr=ag_recv_cw,
                 credit=credit_cw, peer=rgt, up=lft,
                 cols=pl.ds(0, HALF), sgn=-1),
            dict(tx=tx_ccw, rx=rx_ccw, ag=ag_ccw, rss=rs_send_ccw,
                 rsr=rs_recv_ccw, ags=ag_send_ccw, agr=ag_recv_ccw,
                 credit=credit_ccw, peer=lft, up=rgt,
                 cols=pl.ds(HALF, HALF), sgn=+1),
        ]

        def rs_c(d, s):
            return (my_pos + d["sgn"] * s + 2 * N_DEV) % N_DEV

        def ag_c(d, s):
            return (my_pos - d["sgn"] + d["sgn"] * s + 2 * N_DEV) % N_DEV

        def seg(g):
            return pl.ds(g * SROWS, SROWS)

        def chunk_seg(c, g):
            return pl.ds(c * CHUNK + g * SROWS, SROWS)

        barrier_sem = pltpu.get_barrier_semaphore()
        pl.semaphore_signal(barrier_sem, inc=1, device_id=(lft,),
                            device_id_type=pl.DeviceIdType.MESH)
        pl.semaphore_signal(barrier_sem, inc=1, device_id=(rgt,),
                            device_id_type=pl.DeviceIdType.MESH)
        pl.semaphore_wait(barrier_sem, 2)

        def gemm_chunk(c):
            acc = lax.dot_general(
                x_ref[pl.ds(c * CHUNK, CHUNK), :], w_ref[:, :],
                dimension_numbers=(((1,), (0,)), ((), ())),
                preferred_element_type=jnp.int32,
            )
            accb = acc.astype(jnp.bfloat16)
            ag_cw[c] = accb[:, :HALF]
            ag_ccw[c] = accb[:, HALF:]

        def rs_start(d, s, g):
            if s == 0:
                src = d["ag"].at[my_pos, seg(g), :]
            else:
                src = d["tx"].at[s % 2, seg(g), :]
            rd = pltpu.make_async_remote_copy(
                src_ref=src,
                dst_ref=d["rx"].at[s % NSLOT, seg(g), :],
                send_sem=d["rss"].at[s, g],
                recv_sem=d["rsr"].at[s, g],
                device_id=(d["peer"],),
                device_id_type=pl.DeviceIdType.MESH,
            )
            rd.start()
            return rd

        gemm_chunk(my_pos)
        rs_desc = [[[None] * SEG for _ in range(N_DEV - 1)] for _ in D]
        for di, d in enumerate(D):
            for g in range(SEG):
                rs_desc[di][0][g] = rs_start(d, 0, g)
        for k in range(1, N_DEV):
            off = (k + 1) // 2 if k % 2 else k // 2
            sgn = -1 if k % 2 else 1
            gemm_chunk((my_pos + sgn * off + N_DEV) % N_DEV)

        for s in range(N_DEV - 1):
            slot = s % NSLOT
            for g in range(SEG):
                for di, d in enumerate(D):
                    rs_desc[di][s][g].wait_recv()
                    c_in = rs_c(d, s + 1)
                    rx_f32 = d["rx"][slot, seg(g), :].astype(jnp.float32)
                    loc = d["ag"][c_in, seg(g), :].astype(jnp.float32)
                    if s < N_DEV - 2:
                        if g == 0 and s + 1 >= NSLOT:
                            pl.semaphore_wait(d["credit"], 1)
                        if s >= 1:
                            rs_desc[di][s - 1][g].wait_send()
                        d["tx"][(s + 1) % 2, seg(g), :] = (
                            (loc + rx_f32).astype(jnp.bfloat16))
                        rs_desc[di][s + 1][g] = rs_start(d, s + 1, g)
                    else:
                        fin = jnp.maximum((loc + rx_f32) * scale_ref[0], 0.0)
                        out_ref[chunk_seg(c_in, g), d["cols"]] = fin
                        d["ag"][c_in, seg(g), :] = fin.astype(jnp.bfloat16)
            if s <= N_DEV - 2 - NSLOT:
                for d in D:
                    pl.semaphore_signal(d["credit"], inc=1,
                                        device_id=(d["up"],),
                                        device_id_type=pl.DeviceIdType.MESH)
        for di, d in enumerate(D):
            for g in range(SEG):
                rs_desc[di][N_DEV - 3][g].wait_send()
                rs_desc[di][N_DEV - 2][g].wait_send()
        for d in D:
            pl.semaphore_signal(d["credit"], inc=7, device_id=(d["up"],),
                                device_id_type=pl.DeviceIdType.MESH)

        def ag_start(d, s, g):
            ref = d["ag"].at[ag_c(d, s), seg(g), :]
            rd = pltpu.make_async_remote_copy(
                src_ref=ref,
                dst_ref=ref,
                send_sem=d["ags"].at[s, g],
                recv_sem=d["agr"].at[s, g],
                device_id=(d["peer"],),
                device_id_type=pl.DeviceIdType.MESH,
            )
            rd.start()
            return rd

        ag_desc = [[[None] * SEG for _ in range(N_DEV - 1)] for _ in D]
        for di, d in enumerate(D):
            for g in range(SEG):
                ag_desc[di][0][g] = ag_start(d, 0, g)

        for s in range(N_DEV - 1):
            for g in range(SEG):
                for di, d in enumerate(D):
                    ag_desc[di][s][g].wait_recv()
                    c_in = ag_c(d, s + 1)
                    if s < N_DEV - 2:
                        if s + 1 >= 8 and g == 0:
                            pl.semaphore_wait(d["credit"], 1)
                        ag_desc[di][s + 1][g] = ag_start(d, s + 1, g)
                    out_ref[chunk_seg(c_in, g), d["cols"]] = (
                        d["ag"][c_in, seg(g), :].astype(jnp.float32))
        for di, d in enumerate(D):
            for ss in range(N_DEV - 1):
                for g in range(SEG):
                    ag_desc[di][ss][g].wait_send()

    return pl.pallas_call(
        body,
        out_shape=jax.ShapeDtypeStruct((M, N), jnp.float32),
        in_specs=[
            pl.BlockSpec(memory_space=pltpu.VMEM),
            pl.BlockSpec(memory_space=pltpu.VMEM),
            pl.BlockSpec(memory_space=pltpu.SMEM),
            pl.BlockSpec(memory_space=pltpu.SMEM),
            pl.BlockSpec(memory_space=pltpu.SMEM),
            pl.BlockSpec(memory_space=pltpu.SMEM),
        ],
        out_specs=pl.BlockSpec(memory_space=pltpu.VMEM),
        scratch_shapes=[
            pltpu.VMEM((NSLOT, CHUNK, HALF), jnp.bfloat16),
            pltpu.VMEM((NSLOT, CHUNK, HALF), jnp.bfloat16),
            pltpu.VMEM((2, CHUNK, HALF), jnp.bfloat16),
            pltpu.VMEM((2, CHUNK, HALF), jnp.bfloat16),
            pltpu.VMEM((N_DEV, CHUNK, HALF), jnp.bfloat16),
            pltpu.VMEM((N_DEV, CHUNK, HALF), jnp.bfloat16),
            pltpu.SemaphoreType.DMA((N_DEV - 1, SEG)),
            pltpu.SemaphoreType.DMA((N_DEV - 1, SEG)),
            pltpu.SemaphoreType.DMA((N_DEV - 1, SEG)),
            pltpu.SemaphoreType.DMA((N_DEV - 1, SEG)),
            pltpu.SemaphoreType.DMA((N_DEV - 1, SEG)),
            pltpu.SemaphoreType.DMA((N_DEV - 1, SEG)),
            pltpu.SemaphoreType.DMA((N_DEV - 1, SEG)),
            pltpu.SemaphoreType.DMA((N_DEV - 1, SEG)),
            pltpu.SemaphoreType.REGULAR,
            pltpu.SemaphoreType.REGULAR,
        ],
        compiler_params=pltpu.CompilerParams(
            collective_id=0, vmem_limit_bytes=100 * 1024 * 1024
        ),
    )(x, w_mat, scale, pos_s, right_s, left_s)


# device time: 218377 ns/iter; 1.0315x vs baseline; 1.0038x over previous
import jax
import jax.numpy as jnp
from jax import lax
from jax.experimental import pallas as pl
from jax.experimental.pallas import tpu as pltpu

N_DEV = 16
M = 4096
N = 2048
CHUNK = M // N_DEV
HALF = N // 2
NSLOT = 4
SEG = 2
SROWS = CHUNK // SEG

RING = [0, 1, 5, 4, 8, 9, 13, 12, 15, 14, 10, 11, 7, 6, 2, 3]
POS = [0] * N_DEV
for _p, _idx in enumerate(RING):
    POS[_idx] = _p


def kernel(x, w_mat, scale_x, scale_w):
    my_idx = lax.axis_index("i")
    pos = jnp.asarray(POS, jnp.int32)[my_idx]
    ring = jnp.asarray(RING, jnp.int32)
    right = ring[(pos + 1) % N_DEV]
    left = ring[(pos - 1) % N_DEV]

    pos_s = jnp.reshape(pos, (1,)).astype(jnp.int32)
    right_s = jnp.reshape(right, (1,)).astype(jnp.int32)
    left_s = jnp.reshape(left, (1,)).astype(jnp.int32)
    scale = (scale_x * scale_w).astype(jnp.float32)

    def body(x_ref, w_ref, scale_ref, pos_ref, right_ref, left_ref,
             out_ref, rx_cw, rx_ccw, tx_cw, tx_ccw, ag_cw, ag_ccw,
             rs_send_cw, rs_recv_cw, rs_send_ccw, rs_recv_ccw,
             ag_send_cw, ag_recv_cw, ag_send_ccw, ag_recv_ccw,
             credit_cw, credit_ccw):
        my_pos = pos_ref[0]
        rgt = right_ref[0]
        lft = left_ref[0]

        D = [
            dict(tx=tx_cw, rx=rx_cw, ag=ag_cw, rss=rs_send_cw,
                 rsr=rs_recv_cw, ags=ag_send_cw, agr=ag_recv_cw,
                 credit=credit_cw, peer=rgt, up=lft,
                 cols=pl.ds(0, HALF), sgn=-1),
            dict(tx=tx_ccw, rx=rx_ccw, ag=ag_ccw, rss=rs_send_ccw,
                 rsr=rs_recv_ccw, ags=ag_send_ccw, agr=ag_recv_ccw,
                 credit=credit_ccw, peer=lft, up=rgt,
                 cols=pl.ds(HALF, HALF), sgn=+1),
        ]

        def rs_c(d, s):
            return (my_pos + d["sgn"] * s + 2 * N_DEV) % N_DEV

        def ag_c(d, s):
            return (my_pos - d["sgn"] + d["sgn"] * s + 2 * N_DEV) % N_DEV

        def seg(g):
            return pl.ds(g * SROWS, SROWS)

        def chunk_seg(c, g):
            return pl.ds(c * CHUNK + g * SROWS, SROWS)

        barrier_sem = pltpu.get_barrier_semaphore()
        pl.semaphore_signal(barrier_sem, inc=1, device_id=(lft,),
                            device_id_type=pl.DeviceIdType.MESH)
        pl.semaphore_signal(barrier_sem, inc=1, device_id=(rgt,),
                            device_id_type=pl.DeviceIdType.MESH)
        pl.semaphore_wait(barrier_sem, 2)

        def gemm_chunk(c):
            acc = lax.dot_general(
                x_ref[pl.ds(c * CHUNK, CHUNK), :], w_ref[:, :],
                dimension_numbers=(((1,), (0,)), ((), ())),
                preferred_element_type=jnp.int32,
            )
            accb = acc.astype(jnp.bfloat16)
            ag_cw[c] = accb[:, :HALF]
            ag_ccw[c] = accb[:, HALF:]

        def rs_start(d, s, g):
            if s == 0:
                src = d["ag"].at[my_pos, seg(g), :]
            else:
                src = d["tx"].at[s % 2, seg(g), :]
            rd = pltpu.make_async_remote_copy(
                src_ref=src,
                dst_ref=d["rx"].at[s % NSLOT, seg(g), :],
                send_sem=d["rss"].at[s, g],
                recv_sem=d["rsr"].at[s, g],
                device_id=(d["peer"],),
                device_id_type=pl.DeviceIdType.MESH,
            )
            rd.start()
            return rd

        gemm_chunk(my_pos)
        rs_desc = [[[None] * SEG for _ in range(N_DEV - 1)] for _ in D]
        for di, d in enumerate(D):
            for g in range(SEG):
                rs_desc[di][0][g] = rs_start(d, 0, g)
        for k in range(1, N_DEV):
            off = (k + 1) // 2 if k % 2 else k // 2
            sgn = -1 if k % 2 else 1
            gemm_chunk((my_pos + sgn * off + N_DEV) % N_DEV)

        for s in range(N_DEV - 1):
            slot = s % NSLOT
            for g in range(SEG):
                for di, d in enumerate(D):
                    rs_desc[di][s][g].wait_recv()
                    c_in = rs_c(d, s + 1)
                    if s < N_DEV - 2:
                        if g == 0 and s + 1 >= NSLOT:
                            pl.semaphore_wait(d["credit"], 1)
                        if s >= 1:
                            rs_desc[di][s - 1][g].wait_send()
                        d["tx"][(s + 1) % 2, seg(g), :] = (
                            d["ag"][c_in, seg(g), :].astype(jnp.float32)
                            + d["rx"][slot, seg(g), :].astype(jnp.float32)
                        ).astype(jnp.bfloat16)
                        rs_desc[di][s + 1][g] = rs_start(d, s + 1, g)
                    else:
                        fin = jnp.maximum(
                            (d["ag"][c_in, seg(g), :].astype(jnp.float32)
                             + d["rx"][slot, seg(g), :].astype(jnp.float32))
                            * scale_ref[0], 0.0)
                        out_ref[chunk_seg(c_in, g), d["cols"]] = fin
                        d["ag"][c_in, seg(g), :] = fin.astype(jnp.bfloat16)
            if s <= N_DEV - 2 - NSLOT:
                for d in D:
                    pl.semaphore_signal(d["credit"], inc=1,
                                        device_id=(d["up"],),
                                        device_id_type=pl.DeviceIdType.MESH)
        for di, d in enumerate(D):
            for g in range(SEG):
                rs_desc[di][N_DEV - 3][g].wait_send()
                rs_desc[di][N_DEV - 2][g].wait_send()
        for d in D:
            pl.semaphore_signal(d["credit"], inc=7, device_id=(d["up"],),
                                device_id_type=pl.DeviceIdType.MESH)

        def ag_start(d, s, g):
            ref = d["ag"].at[ag_c(d, s), seg(g), :]
            rd = pltpu.make_async_remote_copy(
                src_ref=ref,
                dst_ref=ref,
                send_sem=d["ags"].at[s, g],
                recv_sem=d["agr"].at[s, g],
                device_id=(d["peer"],),
                device_id_type=pl.DeviceIdType.MESH,
            )
            rd.start()
            return rd

        ag_desc = [[[None] * SEG for _ in range(N_DEV - 1)] for _ in D]
        for di, d in enumerate(D):
            for g in range(SEG):
                ag_desc[di][0][g] = ag_start(d, 0, g)

        for s in range(N_DEV - 1):
            for g in range(SEG):
                for di, d in enumerate(D):
                    ag_desc[di][s][g].wait_recv()
                    c_in = ag_c(d, s + 1)
                    if s < N_DEV - 2:
                        if s + 1 >= 8 and g == 0:
                            pl.semaphore_wait(d["credit"], 1)
                        ag_desc[di][s + 1][g] = ag_start(d, s + 1, g)
                    out_ref[chunk_seg(c_in, g), d["cols"]] = (
                        d["ag"][c_in, seg(g), :].astype(jnp.float32))
        for di, d in enumerate(D):
            for ss in range(N_DEV - 1):
                for g in range(SEG):
                    ag_desc[di][ss][g].wait_send()

    return pl.pallas_call(
        body,
        out_shape=jax.ShapeDtypeStruct((M, N), jnp.float32),
        in_specs=[
            pl.BlockSpec(memory_space=pltpu.VMEM),
            pl.BlockSpec(memory_space=pltpu.VMEM),
            pl.BlockSpec(memory_space=pltpu.SMEM),
            pl.BlockSpec(memory_space=pltpu.SMEM),
            pl.BlockSpec(memory_space=pltpu.SMEM),
            pl.BlockSpec(memory_space=pltpu.SMEM),
        ],
        out_specs=pl.BlockSpec(memory_space=pltpu.VMEM),
        scratch_shapes=[
            pltpu.VMEM((NSLOT, CHUNK, HALF), jnp.bfloat16),
            pltpu.VMEM((NSLOT, CHUNK, HALF), jnp.bfloat16),
            pltpu.VMEM((2, CHUNK, HALF), jnp.bfloat16),
            pltpu.VMEM((2, CHUNK, HALF), jnp.bfloat16),
            pltpu.VMEM((N_DEV, CHUNK, HALF), jnp.bfloat16),
            pltpu.VMEM((N_DEV, CHUNK, HALF), jnp.bfloat16),
            pltpu.SemaphoreType.DMA((N_DEV - 1, SEG)),
            pltpu.SemaphoreType.DMA((N_DEV - 1, SEG)),
            pltpu.SemaphoreType.DMA((N_DEV - 1, SEG)),
            pltpu.SemaphoreType.DMA((N_DEV - 1, SEG)),
            pltpu.SemaphoreType.DMA((N_DEV - 1, SEG)),
            pltpu.SemaphoreType.DMA((N_DEV - 1, SEG)),
            pltpu.SemaphoreType.DMA((N_DEV - 1, SEG)),
            pltpu.SemaphoreType.DMA((N_DEV - 1, SEG)),
            pltpu.SemaphoreType.REGULAR,
            pltpu.SemaphoreType.REGULAR,
        ],
        compiler_params=pltpu.CompilerParams(
            collective_id=0, vmem_limit_bytes=100 * 1024 * 1024
        ),
    )(x, w_mat, scale, pos_s, right_s, left_s)
